# baseline (speedup 1.0000x reference)
import functools

import jax
import jax.numpy as jnp
from jax import lax
from jax.experimental import pallas as pl
from jax.experimental.pallas import tpu as pltpu
from jax.experimental.pallas import tpu_sc as plsc

NC = 2
NS = 16
NW = NC * NS
LANES = 16


def _build_emb_kernel(B, L, D, CH):
  W = L // NW
  NCH = (B * W) // CH
  BL = B * L
  mesh = plsc.VectorSubcoreMesh(core_axis_name="c", subcore_axis_name="s")

  @functools.partial(
      pl.kernel,
      out_type=[
          jax.ShapeDtypeStruct((BL, D), jnp.float32),
          jax.ShapeDtypeStruct((BL,), jnp.float32),
      ],
      mesh=mesh,
      scratch_types=[
          pltpu.VMEM((NCH, CH), jnp.int32),
          pltpu.VMEM((W, D), jnp.float32),
          pltpu.VMEM((CH, D), jnp.float32),
          pltpu.VMEM((CH, D), jnp.float32),
          pltpu.VMEM((W,), jnp.float32),
          pltpu.SemaphoreType.DMA,
          pltpu.SemaphoreType.DMA,
          pltpu.SemaphoreType.DMA,
          pltpu.SemaphoreType.DMA,
          pltpu.SemaphoreType.DMA,
      ],
  )
  def emb_kernel(ids_hbm, am_hbm, wte_hbm, wpe_hbm, out_hbm, am_out_hbm,
                 idx_v, wpe_v, rows0, rows1, amb,
                 wsem, g0, g1, s0, s1):
    cid = lax.axis_index("c")
    sid = lax.axis_index("s")
    wid = sid * NC + cid
    l0 = wid * W

    wpe_cp = pltpu.async_copy(wpe_hbm.at[pl.ds(l0, W)], wpe_v, wsem)

    rows = (rows0, rows1)
    gsem = (g0, g1)
    ssem = (s0, s1)
    hpc = W // CH

    def chunk_off(k):
      b, h = divmod(k, hpc)
      return b * L + l0 + h * CH

    def start_gather(k):
      j = k % 2
      pltpu.sync_copy(ids_hbm.at[pl.ds(chunk_off(k), CH)], idx_v.at[k])
      return pltpu.async_copy(wte_hbm.at[idx_v.at[k]], rows[j], gsem[j])

    gathers = [None] * NCH
    scatters = [None] * NCH
    gathers[0] = start_gather(0)
    wpe_cp.wait()

    for k in range(NCH):
      j = k % 2
      gathers[k].wait()
      if k + 1 < NCH:
        if k >= 1:
          scatters[k - 1].wait()
        gathers[k + 1] = start_gather(k + 1)

      wbase = (k % hpc) * CH

      def add_row(r, carry, rows_ref=rows[j], wbase=wbase):
        for cc in range(D // LANES):
          sl = pl.ds(cc * LANES, LANES)
          plsc.addupdate(rows_ref.at[r, sl], wpe_v[wbase + r, sl])
        return carry

      lax.fori_loop(0, CH, add_row, 0)
      scatters[k] = pltpu.async_copy(
          rows[j], out_hbm.at[pl.ds(chunk_off(k), CH)], ssem[j])

    scatters[NCH - 2].wait()
    scatters[NCH - 1].wait()

    for b in range(B):
      off = b * L + l0
      pltpu.sync_copy(am_hbm.at[pl.ds(off, W)], amb)
      for i in range(W // LANES):
        sl = pl.ds(i * LANES, LANES)
        amb[sl] = (1.0 - amb[sl]) * -10000.0
      pltpu.sync_copy(amb, am_out_hbm.at[pl.ds(off, W)])

  return emb_kernel


@jax.jit
def kernel(input_ids, attention_mask, wte, wpe):
  B, L = input_ids.shape
  D = wte.shape[1]
  emb = _build_emb_kernel(B, L, D, CH=32)
  ids_flat = input_ids.reshape(-1).astype(jnp.int32)
  am_flat = attention_mask.reshape(-1).astype(jnp.float32)
  hidden_flat, am_out = emb(ids_flat, am_flat,
                            wte.astype(jnp.float32), wpe.astype(jnp.float32))
  hidden = hidden_flat.reshape(B, L, D)
  am = am_out.reshape(B, 1, 1, L)
  return hidden, am


# speedup vs baseline: 1.3830x; 1.3830x over previous
import functools

import jax
import jax.numpy as jnp
from jax import lax
from jax.experimental import pallas as pl
from jax.experimental.pallas import tpu as pltpu
from jax.experimental.pallas import tpu_sc as plsc

NC = 2
NS = 16
NW = NC * NS
LANES = 16
NBUF = 3


def _build_emb_kernel(B, L, D, CH):
  W = L // NW
  NCH = (B * W) // CH
  mesh = plsc.VectorSubcoreMesh(core_axis_name="c", subcore_axis_name="s")

  @functools.partial(
      pl.kernel,
      out_type=[
          jax.ShapeDtypeStruct((B * L, D), jnp.float32),
          jax.ShapeDtypeStruct((B * L,), jnp.float32),
      ],
      mesh=mesh,
      scratch_types=[
          pltpu.VMEM((B, W), jnp.int32),
          pltpu.VMEM((W, D), jnp.float32),
          [pltpu.VMEM((CH, D), jnp.float32) for _ in range(NBUF)],
          pltpu.VMEM((B, W), jnp.float32),
          pltpu.SemaphoreType.DMA,
          pltpu.SemaphoreType.DMA,
          [pltpu.SemaphoreType.DMA for _ in range(NBUF)],
          [pltpu.SemaphoreType.DMA for _ in range(NBUF)],
      ],
  )
  def emb_kernel(ids_hbm, am_hbm, wte_hbm, wpe_hbm, out_hbm, am_out_hbm,
                 idx_v, wpe_v, rows, amb, wsem, asem, gsem, ssem):
    cid = lax.axis_index("c")
    sid = lax.axis_index("s")
    wid = sid * NC + cid
    l0 = wid * W

    am_cps = [
        pltpu.async_copy(am_hbm.at[pl.ds(b * L + l0, W)], amb.at[b], asem)
        for b in range(B)
    ]
    wpe_cp = pltpu.async_copy(wpe_hbm.at[pl.ds(l0, W)], wpe_v, wsem)
    for b in range(B):
      pltpu.sync_copy(ids_hbm.at[pl.ds(b * L + l0, W)], idx_v.at[b])

    hpc = W // CH

    def start_gather(k):
      b, h = divmod(k, hpc)
      return pltpu.async_copy(
          wte_hbm.at[idx_v.at[b, pl.ds(h * CH, CH)]], rows[k % NBUF],
          gsem[k % NBUF])

    gathers = [None] * NCH
    scatters = [None] * NCH
    for k in range(min(2, NCH)):
      gathers[k] = start_gather(k)
    wpe_cp.wait()

    for k in range(NCH):
      j = k % NBUF
      b, h = divmod(k, hpc)
      gathers[k].wait()

      @plsc.parallel_loop(0, CH)
      def add_row(r, rows_ref=rows[j], wbase=h * CH):
        for cc in range(D // LANES):
          sl = pl.ds(cc * LANES, LANES)
          plsc.addupdate(rows_ref.at[r, sl], wpe_v[wbase + r, sl])

      if k + 2 < NCH:
        if k >= 1:
          scatters[k - 1].wait()
        gathers[k + 2] = start_gather(k + 2)
      scatters[k] = pltpu.async_copy(
          rows[j], out_hbm.at[pl.ds(b * L + l0 + h * CH, CH)], ssem[j])

    for cp in am_cps:
      cp.wait()
    for b in range(B):
      for i in range(W // LANES):
        sl = pl.ds(i * LANES, LANES)
        amb[b, sl] = (1.0 - amb[b, sl]) * -10000.0
      pltpu.sync_copy(amb.at[b], am_out_hbm.at[pl.ds(b * L + l0, W)])

    for k in range(max(0, NCH - 3), NCH):
      scatters[k].wait()

  return emb_kernel


@jax.jit
def kernel(input_ids, attention_mask, wte, wpe):
  B, L = input_ids.shape
  D = wte.shape[1]
  emb = _build_emb_kernel(B, L, D, CH=32)
  hidden_flat, am_out = emb(input_ids.reshape(-1).astype(jnp.int32),
                            attention_mask.reshape(-1).astype(jnp.float32),
                            wte.astype(jnp.float32), wpe.astype(jnp.float32))
  hidden = hidden_flat.reshape(B, L, D)
  am = am_out.reshape(B, 1, 1, L)
  return hidden, am


# speedup vs baseline: 1.3900x; 1.0051x over previous
import functools

import jax
import jax.numpy as jnp
from jax import lax
from jax.experimental import pallas as pl
from jax.experimental.pallas import tpu as pltpu
from jax.experimental.pallas import tpu_sc as plsc

NC = 2
NS = 16
NW = NC * NS
LANES = 16
NBUF = 3


def _build_emb_kernel(B, L, D, CH):
  W = L // NW
  NCH = (B * W) // CH
  mesh = plsc.VectorSubcoreMesh(core_axis_name="c", subcore_axis_name="s")

  @functools.partial(
      pl.kernel,
      out_type=[
          jax.ShapeDtypeStruct((B, L, D), jnp.float32),
          jax.ShapeDtypeStruct((B, 1, 1, L), jnp.float32),
      ],
      mesh=mesh,
      scratch_types=[
          pltpu.VMEM((B, W), jnp.int32),
          pltpu.VMEM((W, D), jnp.float32),
          [pltpu.VMEM((CH, D), jnp.float32) for _ in range(NBUF)],
          pltpu.VMEM((B, W), jnp.float32),
          pltpu.SemaphoreType.DMA,
          pltpu.SemaphoreType.DMA,
          pltpu.SemaphoreType.DMA,
          [pltpu.SemaphoreType.DMA for _ in range(NBUF)],
          [pltpu.SemaphoreType.DMA for _ in range(NBUF)],
      ],
  )
  def emb_kernel(ids_hbm, am_hbm, wte_hbm, wpe_hbm, out_hbm, am_out_hbm,
                 idx_v, wpe_v, rows, amb, wsem, asem, isem, gsem, ssem):
    cid = lax.axis_index("c")
    sid = lax.axis_index("s")
    wid = sid * NC + cid
    l0 = wid * W

    idx_cps = [
        pltpu.async_copy(ids_hbm.at[b, pl.ds(l0, W)], idx_v.at[b], isem)
        for b in range(B)
    ]
    am_cps = [
        pltpu.async_copy(am_hbm.at[b, pl.ds(l0, W)], amb.at[b], asem)
        for b in range(B)
    ]
    wpe_cp = pltpu.async_copy(wpe_hbm.at[pl.ds(l0, W)], wpe_v, wsem)
    for cp in idx_cps:
      cp.wait()

    hpc = W // CH

    def start_gather(k):
      b, h = divmod(k, hpc)
      return pltpu.async_copy(
          wte_hbm.at[idx_v.at[b, pl.ds(h * CH, CH)]], rows[k % NBUF],
          gsem[k % NBUF])

    gathers = [None] * NCH
    scatters = [None] * NCH
    for k in range(min(2, NCH)):
      gathers[k] = start_gather(k)
    wpe_cp.wait()

    for k in range(NCH):
      j = k % NBUF
      b, h = divmod(k, hpc)
      gathers[k].wait()

      @plsc.parallel_loop(0, CH)
      def add_row(r, rows_ref=rows[j], wbase=h * CH):
        for cc in range(D // LANES):
          sl = pl.ds(cc * LANES, LANES)
          plsc.addupdate(rows_ref.at[r, sl], wpe_v[wbase + r, sl])

      if k + 2 < NCH:
        if k >= 1:
          scatters[k - 1].wait()
        gathers[k + 2] = start_gather(k + 2)
      scatters[k] = pltpu.async_copy(
          rows[j], out_hbm.at[b, pl.ds(l0 + h * CH, CH)], ssem[j])

    for cp in am_cps:
      cp.wait()
    for b in range(B):
      for i in range(W // LANES):
        sl = pl.ds(i * LANES, LANES)
        amb[b, sl] = (1.0 - amb[b, sl]) * -10000.0
      pltpu.sync_copy(amb.at[b], am_out_hbm.at[b, 0, 0, pl.ds(l0, W)])

    for k in range(max(0, NCH - 3), NCH):
      scatters[k].wait()

  return emb_kernel


@jax.jit
def kernel(input_ids, attention_mask, wte, wpe):
  B, L = input_ids.shape
  D = wte.shape[1]
  emb = _build_emb_kernel(B, L, D, CH=32)
  hidden, am = emb(input_ids.astype(jnp.int32),
                   attention_mask.astype(jnp.float32),
                   wte.astype(jnp.float32), wpe.astype(jnp.float32))
  return hidden, am
